# baseline (speedup 1.0000x reference)
import functools

import jax
import jax.numpy as jnp
from jax.experimental import pallas as pl

NR = 22350
NP = 22528
BM = 256
C = 51
NO = 150
OC = 151
GI = NP // BM
BN = 2048
GJ = NP // BN


def _prep_body(rel_ref, i0_ref, i1_ref, obj_ref, comb_ref, key2_ref,
               oscore_ref, ocls_ref):
    i = pl.program_id(0)
    po = jax.nn.softmax(obj_ref[...], axis=-1)
    po1 = po[:, 1:]
    om = jnp.max(po1, axis=1, keepdims=True)

    @pl.when(i == 0)
    def _():
        oi = jax.lax.broadcasted_iota(jnp.int32, (NO, OC - 1), 1)
        ocls_ref[...] = jnp.min(jnp.where(po1 == om, oi, OC), axis=1,
                                keepdims=True) + 1
        oscore_ref[...] = om

    kio = jax.lax.broadcasted_iota(jnp.int32, (BM, NO), 1)
    oh0 = (i0_ref[...] == kio).astype(jnp.float32)
    oh1 = (i1_ref[...] == kio).astype(jnp.float32)
    dn = (((1,), (0,)), ((), ()))
    s0 = jax.lax.dot_general(oh0, om, dn, preferred_element_type=jnp.float32)
    s1 = jax.lax.dot_general(oh1, om, dn, preferred_element_type=jnp.float32)

    p = jax.nn.softmax(rel_ref[...], axis=-1)
    p1 = p[:, 1:]
    rmax = jnp.max(p1, axis=1, keepdims=True)
    rio = jax.lax.broadcasted_iota(jnp.int32, (BM, C - 1), 1)
    rcls = jnp.min(jnp.where(p1 == rmax, rio, C), axis=1, keepdims=True) + 1

    triple = (rmax * s0) * s1
    key = jax.lax.bitcast_convert_type(triple, jnp.int32)
    grow = i * BM + jax.lax.broadcasted_iota(jnp.int32, (BM, 1), 0)
    key = jnp.where(grow < NR, key, -1)
    key2_ref[...] = key * 2

    bc = functools.partial(jax.lax.bitcast_convert_type,
                           new_dtype=jnp.float32)
    comb_ref[...] = jnp.concatenate(
        [p, bc(i0_ref[...]), bc(i1_ref[...]), bc(rcls),
         jnp.zeros((BM, 10), jnp.float32)], axis=1)


def _rank_main_body(ki_ref, kj_ref, out_ref, acc_ref):
    i = pl.program_id(0)
    j = pl.program_id(1)

    @pl.when(j == 0)
    def _():
        acc_ref[...] = jnp.zeros_like(acc_ref)

    idiag = i // (BN // BM)

    @pl.when(j != idiag)
    def _():
        tb = jnp.where(j < idiag, 1, 0).astype(jnp.int32)
        kj = kj_ref[...] + tb
        ki = ki_ref[...]
        acc = acc_ref[...]
        for jr in range(16):
            acc = acc + (kj[jr:jr + 1, :] > ki).astype(jnp.int32)
        acc_ref[...] = acc

    @pl.when(j == GJ - 1)
    def _():
        out_ref[...] = jnp.sum(acc_ref[...], axis=1, keepdims=True)


def _rank_diag_body(ki_ref, kj_ref, part_ref, out_ref):
    i = pl.program_id(0)
    ki = ki_ref[...]
    kj = kj_ref[...]
    irow = i * BM + jax.lax.broadcasted_iota(jnp.int32, (BM, 1), 0)
    jbase = ((i // (BN // BM)) * BN
             + jax.lax.broadcasted_iota(jnp.int32, (1, 128), 1))
    acc = jnp.zeros((BM, 128), jnp.int32)
    for jr in range(16):
        tb = ((jbase + jr * 128) < irow).astype(jnp.int32)
        acc = acc + ((kj[jr:jr + 1, :] + tb) > ki).astype(jnp.int32)
    out_ref[...] = part_ref[...] + jnp.sum(acc, axis=1, keepdims=True)


def _apply_perm(comb, ranks):
    return jnp.zeros((NP, 64), jnp.float32).at[ranks].set(
        comb, mode="drop", unique_indices=True)


def kernel(rel_logit, obj_logit, rel_pair_idx):
    pad = NP - NR
    rel_p = jnp.pad(rel_logit, ((0, pad), (0, 0)))
    i0 = jnp.pad(rel_pair_idx[:, 0], (0, pad)).reshape(NP, 1)
    i1 = jnp.pad(rel_pair_idx[:, 1], (0, pad)).reshape(NP, 1)

    comb, key2, oscore, ocls = pl.pallas_call(
        _prep_body,
        grid=(GI,),
        in_specs=[
            pl.BlockSpec((BM, C), lambda i: (i, 0)),
            pl.BlockSpec((BM, 1), lambda i: (i, 0)),
            pl.BlockSpec((BM, 1), lambda i: (i, 0)),
            pl.BlockSpec((NO, OC), lambda i: (0, 0)),
        ],
        out_specs=[
            pl.BlockSpec((BM, 64), lambda i: (i, 0)),
            pl.BlockSpec((BM, 1), lambda i: (i, 0)),
            pl.BlockSpec((NO, 1), lambda i: (0, 0)),
            pl.BlockSpec((NO, 1), lambda i: (0, 0)),
        ],
        out_shape=[
            jax.ShapeDtypeStruct((NP, 64), jnp.float32),
            jax.ShapeDtypeStruct((NP, 1), jnp.int32),
            jax.ShapeDtypeStruct((NO, 1), jnp.float32),
            jax.ShapeDtypeStruct((NO, 1), jnp.int32),
        ],
    )(rel_p, i0, i1, obj_logit)

    ocp = jax.nn.softmax(obj_logit, axis=-1)
    osc = jnp.max(ocp[:, 1:], axis=1)
    rcp = jax.nn.softmax(rel_logit, axis=-1)
    rsc = jnp.max(rcp[:, 1:], axis=1)
    trip = rsc * osc[rel_pair_idx[:, 0]] * osc[rel_pair_idx[:, 1]]
    kx = jax.lax.bitcast_convert_type(trip, jnp.int32)
    key2 = (jnp.pad(kx, (0, NP - NR), constant_values=-1) * 2).reshape(NP, 1)

    key_rows = key2.reshape(NP // 128, 128)
    part = pl.pallas_call(
        _rank_main_body,
        grid=(GI, GJ),
        in_specs=[
            pl.BlockSpec((BM, 1), lambda i, j: (i, 0)),
            pl.BlockSpec((BN // 128, 128), lambda i, j: (j, 0)),
        ],
        out_specs=pl.BlockSpec((BM, 1), lambda i, j: (i, 0)),
        out_shape=jax.ShapeDtypeStruct((NP, 1), jnp.int32),
        scratch_shapes=[pltpu_vmem((BM, 128), jnp.int32)],
        compiler_params=_tc_params(),
    )(key2, key_rows)
    ranks = pl.pallas_call(
        _rank_diag_body,
        grid=(GI,),
        in_specs=[
            pl.BlockSpec((BM, 1), lambda i: (i, 0)),
            pl.BlockSpec((BN // 128, 128), lambda i: (i // (BN // BM), 0)),
            pl.BlockSpec((BM, 1), lambda i: (i, 0)),
        ],
        out_specs=pl.BlockSpec((BM, 1), lambda i: (i, 0)),
        out_shape=jax.ShapeDtypeStruct((NP, 1), jnp.int32),
    )(key2, key_rows, part)

    sorted_comb = _apply_perm(comb, ranks.reshape(NP))[:NR]
    prob = sorted_comb[:, :C]
    bci = functools.partial(jax.lax.bitcast_convert_type,
                            new_dtype=jnp.int32)
    pair = bci(sorted_comb[:, C:C + 2])
    labels = bci(sorted_comb[:, C + 2])
    return (ocls.reshape(NO), oscore.reshape(NO), pair, prob, labels)


def pltpu_vmem(shape, dtype):
    from jax.experimental.pallas import tpu as pltpu
    return pltpu.VMEM(shape, dtype)


def _tc_params():
    from jax.experimental.pallas import tpu as pltpu
    return pltpu.CompilerParams(
        dimension_semantics=("arbitrary", "arbitrary"))


# speedup vs baseline: 1.0380x; 1.0380x over previous
import functools

import jax
import jax.numpy as jnp
from jax.experimental import pallas as pl

NR = 22350
NP = 22528
BM = 256
C = 51
NO = 150
OC = 151
GI = NP // BM
BN = 2048
GJ = NP // BN


def _prep_body(rel_ref, i0_ref, i1_ref, obj_ref, comb_ref,
               oscore_ref, ocls_ref):
    i = pl.program_id(0)
    po = jax.nn.softmax(obj_ref[...], axis=-1)
    po1 = po[:, 1:]
    om = jnp.max(po1, axis=1, keepdims=True)

    @pl.when(i == 0)
    def _():
        oi = jax.lax.broadcasted_iota(jnp.int32, (NO, OC - 1), 1)
        ocls_ref[...] = jnp.min(jnp.where(po1 == om, oi, OC), axis=1,
                                keepdims=True) + 1
        oscore_ref[...] = om

    p = jax.nn.softmax(rel_ref[...], axis=-1)
    p1 = p[:, 1:]
    rmax = jnp.max(p1, axis=1, keepdims=True)
    rio = jax.lax.broadcasted_iota(jnp.int32, (BM, C - 1), 1)
    rcls = jnp.min(jnp.where(p1 == rmax, rio, C), axis=1, keepdims=True) + 1

    bc = functools.partial(jax.lax.bitcast_convert_type,
                           new_dtype=jnp.float32)
    comb_ref[...] = jnp.concatenate(
        [p, bc(i0_ref[...]), bc(i1_ref[...]), bc(rcls),
         jnp.zeros((BM, 74), jnp.float32)], axis=1)


def _rank_main_body(ki_ref, kj_ref, out_ref, acc_ref):
    i = pl.program_id(0)
    j = pl.program_id(1)

    @pl.when(j == 0)
    def _():
        acc_ref[...] = jnp.zeros_like(acc_ref)

    idiag = i // (BN // BM)

    @pl.when(j != idiag)
    def _():
        tb = jnp.where(j < idiag, 1, 0).astype(jnp.int32)
        kj = kj_ref[...] + tb
        ki = ki_ref[...]
        acc = acc_ref[...]
        for jr in range(16):
            acc = acc + (kj[jr:jr + 1, :] > ki).astype(jnp.int32)
        acc_ref[...] = acc

    @pl.when(j == GJ - 1)
    def _():
        out_ref[...] = jnp.sum(acc_ref[...], axis=1, keepdims=True)


def _rank_diag_body(ki_ref, kj_ref, part_ref, out_ref):
    i = pl.program_id(0)
    ki = ki_ref[...]
    kj = kj_ref[...]
    irow = i * BM + jax.lax.broadcasted_iota(jnp.int32, (BM, 1), 0)
    jbase = ((i // (BN // BM)) * BN
             + jax.lax.broadcasted_iota(jnp.int32, (1, 128), 1))
    acc = jnp.zeros((BM, 128), jnp.int32)
    for jr in range(16):
        tb = ((jbase + jr * 128) < irow).astype(jnp.int32)
        acc = acc + ((kj[jr:jr + 1, :] + tb) > ki).astype(jnp.int32)
    out_ref[...] = part_ref[...] + jnp.sum(acc, axis=1, keepdims=True)


NWORK = 32
RPW = NP // NWORK
IROWS = 8
IW = RPW // IROWS


def _sc_scatter():
    from jax.experimental.pallas import tpu as pltpu
    from jax.experimental.pallas import tpu_sc as plsc
    from jax import lax

    mesh = plsc.VectorSubcoreMesh(core_axis_name="c", subcore_axis_name="s")

    @functools.partial(
        pl.kernel, mesh=mesh,
        out_type=jax.ShapeDtypeStruct((NP, 128), jnp.float32),
        scratch_types=[
            pltpu.VMEM((IROWS, IW), jnp.int32),
            pltpu.VMEM((RPW, 128), jnp.float32),
            pltpu.SemaphoreType.DMA,
        ],
    )
    def scat(comb_hbm, ranks_hbm, out_hbm, idx_v, rows_v, sem):
        wid = lax.axis_index("s") * 2 + lax.axis_index("c")
        pltpu.sync_copy(ranks_hbm.at[pl.ds(wid * IROWS, IROWS)], idx_v)
        pltpu.sync_copy(comb_hbm.at[pl.ds(wid * RPW, RPW)], rows_v)
        copies = [
            pltpu.async_copy(rows_v.at[pl.ds(jj * IW, IW)],
                             out_hbm.at[idx_v.at[jj]], sem)
            for jj in range(IROWS)
        ]
        for cp in copies:
            cp.wait()

    return scat


def _apply_perm(comb, ranks):
    return _sc_scatter()(comb, ranks.reshape(NWORK * IROWS, IW))


def kernel(rel_logit, obj_logit, rel_pair_idx):
    pad = NP - NR
    rel_p = jnp.pad(rel_logit, ((0, pad), (0, 0)))
    i0 = jnp.pad(rel_pair_idx[:, 0], (0, pad)).reshape(NP, 1)
    i1 = jnp.pad(rel_pair_idx[:, 1], (0, pad)).reshape(NP, 1)

    comb, oscore, ocls = pl.pallas_call(
        _prep_body,
        grid=(GI,),
        in_specs=[
            pl.BlockSpec((BM, C), lambda i: (i, 0)),
            pl.BlockSpec((BM, 1), lambda i: (i, 0)),
            pl.BlockSpec((BM, 1), lambda i: (i, 0)),
            pl.BlockSpec((NO, OC), lambda i: (0, 0)),
        ],
        out_specs=[
            pl.BlockSpec((BM, 128), lambda i: (i, 0)),
            pl.BlockSpec((NO, 1), lambda i: (0, 0)),
            pl.BlockSpec((NO, 1), lambda i: (0, 0)),
        ],
        out_shape=[
            jax.ShapeDtypeStruct((NP, 128), jnp.float32),
            jax.ShapeDtypeStruct((NO, 1), jnp.float32),
            jax.ShapeDtypeStruct((NO, 1), jnp.int32),
        ],
    )(rel_p, i0, i1, obj_logit)

    ocp = jax.nn.softmax(obj_logit, axis=-1)
    osc = jnp.max(ocp[:, 1:], axis=1)
    rcp = jax.nn.softmax(rel_logit, axis=-1)
    rsc = jnp.max(rcp[:, 1:], axis=1)
    trip = rsc * osc[rel_pair_idx[:, 0]] * osc[rel_pair_idx[:, 1]]
    kx = jax.lax.bitcast_convert_type(trip, jnp.int32)
    key2 = (jnp.pad(kx, (0, NP - NR), constant_values=-1) * 2).reshape(NP, 1)

    key_rows = key2.reshape(NP // 128, 128)
    part = pl.pallas_call(
        _rank_main_body,
        grid=(GI, GJ),
        in_specs=[
            pl.BlockSpec((BM, 1), lambda i, j: (i, 0)),
            pl.BlockSpec((BN // 128, 128), lambda i, j: (j, 0)),
        ],
        out_specs=pl.BlockSpec((BM, 1), lambda i, j: (i, 0)),
        out_shape=jax.ShapeDtypeStruct((NP, 1), jnp.int32),
        scratch_shapes=[pltpu_vmem((BM, 128), jnp.int32)],
        compiler_params=_tc_params(),
    )(key2, key_rows)
    ranks = pl.pallas_call(
        _rank_diag_body,
        grid=(GI,),
        in_specs=[
            pl.BlockSpec((BM, 1), lambda i: (i, 0)),
            pl.BlockSpec((BN // 128, 128), lambda i: (i // (BN // BM), 0)),
            pl.BlockSpec((BM, 1), lambda i: (i, 0)),
        ],
        out_specs=pl.BlockSpec((BM, 1), lambda i: (i, 0)),
        out_shape=jax.ShapeDtypeStruct((NP, 1), jnp.int32),
    )(key2, key_rows, part)

    sorted_comb = _apply_perm(comb, ranks.reshape(NP))[:NR]
    prob = sorted_comb[:, :C]
    bci = functools.partial(jax.lax.bitcast_convert_type,
                            new_dtype=jnp.int32)
    pair = bci(sorted_comb[:, C:C + 2])
    labels = bci(sorted_comb[:, C + 2])
    return (ocls.reshape(NO), oscore.reshape(NO), pair, prob, labels)


def pltpu_vmem(shape, dtype):
    from jax.experimental.pallas import tpu as pltpu
    return pltpu.VMEM(shape, dtype)


def _tc_params():
    from jax.experimental.pallas import tpu as pltpu
    return pltpu.CompilerParams(
        dimension_semantics=("arbitrary", "arbitrary"))
